# baseline (speedup 1.0000x reference)
import functools

import jax
import jax.numpy as jnp
from jax import lax
from jax.experimental import pallas as pl
from jax.experimental.pallas import tpu as pltpu
from jax.experimental.pallas import tpu_sc as plsc

B = 1024
L_POS = 20
L_SESS = 50
NUM_CLASSES = 100000
NEG_CAND = 256

def _cand_pool():
    return jax.vmap(lambda k: jax.random.randint(k, (NEG_CAND,), 0, NUM_CLASSES))(
        jax.random.split(jax.random.key(42), B)
    )

_ROWS_PER_TILE = B // 32
_NEG_CHUNKS = _ROWS_PER_TILE * B // 128
_POS_CHUNKS = _ROWS_PER_TILE * L_POS // 128


def _neg_select_body(lab_ref, ses_ref, cand_ref, neg_ref):
    lab = lab_ref[...]
    ses = ses_ref[...]
    cnd = cand_ref[...]
    bad = jnp.zeros(cnd.shape, jnp.bool_)
    for t in range(L_POS):
        bad = bad | (cnd == lab[:, t][:, None])
    for t in range(L_SESS):
        bad = bad | (cnd == ses[:, t][:, None])
    ii = lax.broadcasted_iota(jnp.int32, cnd.shape, 1)
    score = jnp.where(bad, jnp.int32(1 << 20), ii)
    first = jnp.min(score, axis=1, keepdims=True)
    sel = jnp.where(first >= NEG_CAND, 0, first)
    neg_ref[...] = jnp.sum(jnp.where(ii == sel, cnd, 0), axis=1, keepdims=True)


def _neg_select(labels, sessions, cand):
    blk = 128
    return pl.pallas_call(
        _neg_select_body,
        grid=(B // blk,),
        in_specs=[
            pl.BlockSpec((blk, L_POS), lambda i: (i, 0)),
            pl.BlockSpec((blk, L_SESS), lambda i: (i, 0)),
            pl.BlockSpec((blk, NEG_CAND), lambda i: (i, 0)),
        ],
        out_specs=pl.BlockSpec((blk, 1), lambda i: (i, 0)),
        out_shape=jax.ShapeDtypeStruct((B, 1), jnp.int32),
    )(labels, sessions, cand)


def _sc_gather_body(outflat, neg_hbm, labflat, gneg_hbm, gpos_hbm,
                    neg_v, lab_v, idx_v, dat_v, pidx_v, pdat_v, sem):
    wid = lax.axis_index("s") * 2 + lax.axis_index("c")
    row0 = wid * _ROWS_PER_TILE

    pltpu.sync_copy(neg_hbm, neg_v)
    pltpu.sync_copy(labflat.at[pl.ds(wid * (_ROWS_PER_TILE * L_POS),
                                     _ROWS_PER_TILE * L_POS)], lab_v)

    def body_cc(cc, carry):
        r = cc // 8
        base_v = jnp.full((16,), (row0 + r) * NUM_CLASSES, jnp.int32)
        off = (cc % 8) * 128
        for v in range(8):
            idx_v[cc, pl.ds(v * 16, 16)] = (
                neg_v[pl.ds(off + v * 16, 16)] + base_v)
        return carry
    lax.fori_loop(0, _NEG_CHUNKS, body_cc, 0)

    lane = lax.iota(jnp.int32, 16)
    row0_v = jnp.full((16,), row0, jnp.int32)
    one_v = jnp.full((16,), 1, jnp.int32)
    zero_v = jnp.full((16,), 0, jnp.int32)
    cls_v = jnp.full((16,), NUM_CLASSES, jnp.int32)
    for c in range(_ROWS_PER_TILE * L_POS // 16):
        k0 = c * 16
        r0 = k0 // L_POS
        bound = (r0 + 1) * L_POS
        inc = jnp.where(lane >= jnp.full((16,), bound - k0, jnp.int32),
                        one_v, zero_v)
        row = row0_v + jnp.full((16,), r0, jnp.int32) + inc
        pid = row * cls_v + lab_v[pl.ds(k0, 16)]
        pidx_v[c // 8, pl.ds((c % 8) * 16, 16)] = pid

    def fire(cc, carry):
        pltpu.async_copy(outflat.at[idx_v.at[cc]], dat_v.at[cc], sem)
        return carry
    lax.fori_loop(0, _NEG_CHUNKS, fire, 0)
    for c in range(_POS_CHUNKS):
        pltpu.async_copy(outflat.at[pidx_v.at[c]], pdat_v.at[c], sem)

    def drain(cc, carry):
        pltpu.make_async_copy(outflat.at[idx_v.at[cc]], dat_v.at[cc], sem).wait()
        return carry
    lax.fori_loop(0, _NEG_CHUNKS, drain, 0)
    for c in range(_POS_CHUNKS):
        pltpu.make_async_copy(outflat.at[pidx_v.at[c]], pdat_v.at[c], sem).wait()
    pltpu.sync_copy(dat_v, gneg_hbm.at[wid])
    pltpu.sync_copy(pdat_v, gpos_hbm.at[wid])


@functools.cache
def _make_sc_gather():
    return pl.kernel(
        _sc_gather_body,
        mesh=plsc.VectorSubcoreMesh(core_axis_name="c", subcore_axis_name="s"),
        out_type=[
            jax.ShapeDtypeStruct((32, _NEG_CHUNKS, 128), jnp.float32),
            jax.ShapeDtypeStruct((32, _POS_CHUNKS, 128), jnp.float32),
        ],
        scratch_types=[
            pltpu.VMEM((B,), jnp.int32),
            pltpu.VMEM((_ROWS_PER_TILE * L_POS,), jnp.int32),
            pltpu.VMEM((_NEG_CHUNKS, 128), jnp.int32),
            pltpu.VMEM((_NEG_CHUNKS, 128), jnp.float32),
            pltpu.VMEM((_POS_CHUNKS, 128), jnp.int32),
            pltpu.VMEM((_POS_CHUNKS, 128), jnp.float32),
            pltpu.SemaphoreType.DMA,
        ],
    )


def _loss_body(gn_ref, gp_ref, out_ref):
    xn = gn_ref[...]
    sn = jax.nn.sigmoid(xn)
    fn = -jnp.log(1.0 - sn + 1e-10)
    xp = gp_ref[...]
    sp = jax.nn.sigmoid(xp)
    gp = -jnp.log(sp + 1e-10)
    total = jnp.sum(fn) / B + jnp.sum(gp) / (L_POS * B)
    out_ref[...] = jnp.reshape(total, (1, 1))


def _loss_reduce(gneg, gpos):
    return pl.pallas_call(
        _loss_body,
        out_shape=jax.ShapeDtypeStruct((1, 1), jnp.float32),
    )(gneg, gpos)


def kernel(outputs, labels, sessions):
    cand = _cand_pool()
    neg = _neg_select(labels, sessions, cand).reshape(B)
    outflat = outputs.reshape(-1)
    labflat = labels.reshape(-1)
    gneg, gpos = _make_sc_gather()(outflat, neg, labflat)
    loss = _loss_reduce(gneg.reshape(32 * _NEG_CHUNKS, 128),
                        gpos.reshape(32 * _POS_CHUNKS, 128))
    return loss[0, 0]


# speedup vs baseline: 12.0097x; 12.0097x over previous
import functools

import jax
import jax.numpy as jnp
from jax import lax
from jax.experimental import pallas as pl
from jax.experimental.pallas import tpu as pltpu
from jax.experimental.pallas import tpu_sc as plsc

B = 1024
L_POS = 20
L_SESS = 50
NUM_CLASSES = 100000
NEG_CAND = 256

_NTILES = 32
_COLS_PER_TILE = B // _NTILES
_ROWS_PER_TILE = B // _NTILES
_PPT = _ROWS_PER_TILE * L_POS
_NEG_ROWS = _COLS_PER_TILE * 8
_POS_CHUNKS = _PPT // 128


def _cand_pool():
    return jax.vmap(lambda k: jax.random.randint(k, (NEG_CAND,), 0, NUM_CLASSES))(
        jax.random.split(jax.random.key(42), B)
    )


def _neg_select_body(lab_ref, ses_ref, cand_ref, neg_ref):
    lab = lab_ref[...]
    ses = ses_ref[...]
    cnd = cand_ref[...]
    bad = jnp.zeros(cnd.shape, jnp.bool_)
    for t in range(L_POS):
        bad = bad | (cnd == lab[:, t][:, None])
    for t in range(L_SESS):
        bad = bad | (cnd == ses[:, t][:, None])
    ii = lax.broadcasted_iota(jnp.int32, cnd.shape, 1)
    score = jnp.where(bad, jnp.int32(1 << 20), ii)
    first = jnp.min(score, axis=1, keepdims=True)
    sel = jnp.where(first >= NEG_CAND, 0, first)
    neg_ref[...] = jnp.sum(jnp.where(ii == sel, cnd, 0), axis=1, keepdims=True)


def _neg_select(labels, sessions, cand):
    blk = 128
    return pl.pallas_call(
        _neg_select_body,
        grid=(B // blk,),
        in_specs=[
            pl.BlockSpec((blk, L_POS), lambda i: (i, 0)),
            pl.BlockSpec((blk, L_SESS), lambda i: (i, 0)),
            pl.BlockSpec((blk, NEG_CAND), lambda i: (i, 0)),
        ],
        out_specs=pl.BlockSpec((blk, 1), lambda i: (i, 0)),
        out_shape=jax.ShapeDtypeStruct((B, 1), jnp.int32),
    )(labels, sessions, cand)


def _sc_gather_body(view, neg_hbm, labflat, gneg_hbm, gpos_hbm,
                    negw_v, lab_v, idx_v, dat_v, pidx_v, pdat_v, sem):
    wid = lax.axis_index("s") * 2 + lax.axis_index("c")

    pltpu.sync_copy(neg_hbm.at[pl.ds(wid * _COLS_PER_TILE, _COLS_PER_TILE)],
                    negw_v.at[pl.ds(0, _COLS_PER_TILE)])
    pltpu.sync_copy(labflat.at[pl.ds(wid * _PPT, _PPT)], lab_v)

    three = jnp.full((16,), 3, jnp.int32)
    six = jnp.full((16,), 6, jnp.int32)
    seven = jnp.full((16,), 7, jnp.int32)
    lane = lax.iota(jnp.int32, 16)

    for jc in range(_COLS_PER_TILE // 16):
        c = negw_v[pl.ds(jc * 16, 16)]
        base = lax.shift_left(lax.shift_right_logical(c, three), six) + (c & seven)
        for it in range(8):
            p0 = it * _COLS_PER_TILE + jc * 16
            idx_v[p0 // 128, pl.ds(p0 % 128, 16)] = (
                base + jnp.full((16,), it * 8, jnp.int32))

    for c40 in range(_PPT // 16):
        k0 = c40 * 16
        r0 = k0 // L_POS
        bound = (r0 + 1) * L_POS
        inc = jnp.where(lane >= jnp.full((16,), bound - k0, jnp.int32),
                        jnp.full((16,), 1, jnp.int32),
                        jnp.full((16,), 0, jnp.int32))
        ivec = jnp.full((16,), wid * _ROWS_PER_TILE + r0, jnp.int32) + inc
        cvec = lab_v[pl.ds(k0, 16)]
        rvec = (lax.shift_left(lax.shift_right_logical(cvec, three), six)
                + lax.shift_left(lax.shift_right_logical(ivec, seven), three)
                + (cvec & seven))
        pidx_v[c40 // 8, pl.ds((c40 % 8) * 16, 16)] = rvec

    cps = [pltpu.async_copy(view.at[idx_v.at[q]], dat_v.at[q], sem)
           for q in range(_NEG_ROWS // 128)]
    cps += [pltpu.async_copy(view.at[pidx_v.at[q]], pdat_v.at[q], sem)
            for q in range(_POS_CHUNKS)]
    for cp in cps:
        cp.wait()
    pltpu.sync_copy(dat_v, gneg_hbm.at[wid])
    pltpu.sync_copy(pdat_v, gpos_hbm.at[wid])


@functools.cache
def _make_sc_gather():
    return pl.kernel(
        _sc_gather_body,
        mesh=plsc.VectorSubcoreMesh(core_axis_name="c", subcore_axis_name="s"),
        out_type=[
            jax.ShapeDtypeStruct((_NTILES, _NEG_ROWS // 128, 128, 128),
                                 jnp.float32),
            jax.ShapeDtypeStruct((_NTILES, _POS_CHUNKS, 128, 128),
                                 jnp.float32),
        ],
        scratch_types=[
            pltpu.VMEM((128,), jnp.int32),
            pltpu.VMEM((_PPT,), jnp.int32),
            pltpu.VMEM((_NEG_ROWS // 128, 128), jnp.int32),
            pltpu.VMEM((_NEG_ROWS // 128, 128, 128), jnp.float32),
            pltpu.VMEM((_POS_CHUNKS, 128), jnp.int32),
            pltpu.VMEM((_POS_CHUNKS, 128, 128), jnp.float32),
            pltpu.SemaphoreType.DMA,
        ],
    )


_GRID = 8
_NB = B * B // 128 // _GRID
_PB = B * L_POS // _GRID // 128


def _loss_body(gn_ref, gp_ref, out_ref, acc_ref):
    step = pl.program_id(0)

    xn = gn_ref[...]
    sn = jax.nn.sigmoid(xn)
    neg_sum = jnp.sum(-jnp.log(1.0 - sn + 1e-10))

    praw = gp_ref[...]
    rr = lax.broadcasted_iota(jnp.int32, praw.shape, 0) + step * (_PB * 128)
    tgt = (rr // L_POS) & 127
    ll = lax.broadcasted_iota(jnp.int32, praw.shape, 1)
    xp = jnp.sum(jnp.where(tgt == ll, praw, 0.0), axis=1, keepdims=True)
    sp = jax.nn.sigmoid(xp)
    pos_sum = jnp.sum(-jnp.log(sp + 1e-10))

    part = neg_sum / B + pos_sum / (L_POS * B)

    @pl.when(step == 0)
    def _():
        acc_ref[0] = 0.0

    acc_ref[0] += part

    @pl.when(step == _GRID - 1)
    def _():
        out_ref[...] = jnp.reshape(acc_ref[0], (1, 1))


def _loss_reduce(gneg, gpos):
    return pl.pallas_call(
        _loss_body,
        grid=(_GRID,),
        in_specs=[
            pl.BlockSpec((_NB, 128), lambda i: (i, 0)),
            pl.BlockSpec((_PB * 128, 128), lambda i: (i, 0)),
        ],
        out_specs=pl.BlockSpec((1, 1), lambda i: (0, 0)),
        out_shape=jax.ShapeDtypeStruct((1, 1), jnp.float32),
        scratch_shapes=[pltpu.SMEM((1,), jnp.float32)],
    )(gneg, gpos)


def kernel(outputs, labels, sessions):
    cand = _cand_pool()
    neg = _neg_select(labels, sessions, cand).reshape(B)
    view = (jnp.transpose(outputs)
            .reshape(NUM_CLASSES // 8, 8, 8, 128)
            .transpose(0, 2, 1, 3)
            .reshape(NUM_CLASSES * 8, 128))
    gneg, gpos = _make_sc_gather()(view, neg, labels.reshape(-1))
    loss = _loss_reduce(gneg.reshape(B * B // 128, 128),
                        gpos.reshape(B * L_POS, 128))
    return loss[0, 0]


# speedup vs baseline: 15.3285x; 1.2763x over previous
import functools

import jax
import jax.numpy as jnp
from jax import lax
from jax.experimental import pallas as pl
from jax.experimental.pallas import tpu as pltpu
from jax.experimental.pallas import tpu_sc as plsc

B = 1024
L_POS = 20
L_SESS = 50
NUM_CLASSES = 100000
NEG_CAND = 256

_NTILES = 32
_COLS_PER_TILE = B // _NTILES
_ROWS_PER_TILE = B // _NTILES
_PPT = _ROWS_PER_TILE * L_POS
_NEG_ROWS = _COLS_PER_TILE * 8
_POS_CHUNKS = _PPT // 128


def _cand_pool():
    return jax.vmap(lambda k: jax.random.randint(k, (NEG_CAND,), 0, NUM_CLASSES))(
        jax.random.split(jax.random.key(42), B)
    )


def _neg_select_body(lab_ref, ses_ref, cand_ref, neg_ref):
    lab = lab_ref[...]
    ses = ses_ref[...]
    cnd = cand_ref[...]
    bad = jnp.zeros(cnd.shape, jnp.bool_)
    for t in range(L_POS):
        bad = bad | (cnd == lab[:, t][:, None])
    for t in range(L_SESS):
        bad = bad | (cnd == ses[:, t][:, None])
    ii = lax.broadcasted_iota(jnp.int32, cnd.shape, 1)
    score = jnp.where(bad, jnp.int32(1 << 20), ii)
    first = jnp.min(score, axis=1, keepdims=True)
    sel = jnp.where(first >= NEG_CAND, 0, first)
    neg_ref[...] = jnp.sum(jnp.where(ii == sel, cnd, 0), axis=1, keepdims=True)


def _neg_select(labels, sessions, cand):
    blk = 128
    return pl.pallas_call(
        _neg_select_body,
        grid=(B // blk,),
        in_specs=[
            pl.BlockSpec((blk, L_POS), lambda i: (i, 0)),
            pl.BlockSpec((blk, L_SESS), lambda i: (i, 0)),
            pl.BlockSpec((blk, NEG_CAND), lambda i: (i, 0)),
        ],
        out_specs=pl.BlockSpec((blk, 1), lambda i: (i, 0)),
        out_shape=jax.ShapeDtypeStruct((B, 1), jnp.int32),
    )(labels, sessions, cand)


def _sc_neg_body(view, neg_hbm, gneg_hbm, negw_v, idx_v, dat_v, sem):
    wid = lax.axis_index("s") * 2 + lax.axis_index("c")

    pltpu.sync_copy(neg_hbm.at[pl.ds(wid * _COLS_PER_TILE, _COLS_PER_TILE)],
                    negw_v.at[pl.ds(0, _COLS_PER_TILE)])

    three = jnp.full((16,), 3, jnp.int32)
    six = jnp.full((16,), 6, jnp.int32)
    seven = jnp.full((16,), 7, jnp.int32)

    for jc in range(_COLS_PER_TILE // 16):
        c = negw_v[pl.ds(jc * 16, 16)]
        base = lax.shift_left(lax.shift_right_logical(c, three), six) + (c & seven)
        for it in range(8):
            p0 = it * _COLS_PER_TILE + jc * 16
            idx_v[p0 // 128, pl.ds(p0 % 128, 16)] = (
                base + jnp.full((16,), it * 8, jnp.int32))

    cps = [pltpu.async_copy(view.at[idx_v.at[q]], dat_v.at[q], sem)
           for q in range(_NEG_ROWS // 128)]
    for cp in cps:
        cp.wait()
    pltpu.sync_copy(dat_v, gneg_hbm.at[wid])


def _sc_pos_body(view16, labflat, gpos_hbm, lab_v, pidx_v, pdat_v, sem):
    wid = lax.axis_index("s") * 2 + lax.axis_index("c")

    pltpu.sync_copy(labflat.at[pl.ds(wid * _PPT, _PPT)], lab_v)

    three = jnp.full((16,), 3, jnp.int32)
    four = jnp.full((16,), 4, jnp.int32)
    six = jnp.full((16,), 6, jnp.int32)
    seven = jnp.full((16,), 7, jnp.int32)
    m127 = jnp.full((16,), 127, jnp.int32)
    lane = lax.iota(jnp.int32, 16)

    for c40 in range(_PPT // 16):
        k0 = c40 * 16
        r0 = k0 // L_POS
        bound = (r0 + 1) * L_POS
        inc = jnp.where(lane >= jnp.full((16,), bound - k0, jnp.int32),
                        jnp.full((16,), 1, jnp.int32),
                        jnp.full((16,), 0, jnp.int32))
        ivec = jnp.full((16,), wid * _ROWS_PER_TILE + r0, jnp.int32) + inc
        cvec = lab_v[pl.ds(k0, 16)]
        rvec = (lax.shift_left(lax.shift_right_logical(cvec, three), six)
                + lax.shift_left(lax.shift_right_logical(ivec, seven), three)
                + (cvec & seven))
        gvec = (lax.shift_left(rvec, three)
                + lax.shift_right_logical(ivec & m127, four))
        pidx_v[c40 // 8, pl.ds((c40 % 8) * 16, 16)] = gvec

    cps = [pltpu.async_copy(view16.at[pidx_v.at[q]], pdat_v.at[q], sem)
           for q in range(_POS_CHUNKS)]
    for cp in cps:
        cp.wait()
    pltpu.sync_copy(pdat_v, gpos_hbm.at[wid])


@functools.cache
def _make_sc_neg():
    return pl.kernel(
        _sc_neg_body,
        mesh=plsc.VectorSubcoreMesh(core_axis_name="c", subcore_axis_name="s"),
        out_type=[
            jax.ShapeDtypeStruct((_NTILES, _NEG_ROWS // 128, 128, 128),
                                 jnp.float32),
        ],
        scratch_types=[
            pltpu.VMEM((128,), jnp.int32),
            pltpu.VMEM((_NEG_ROWS // 128, 128), jnp.int32),
            pltpu.VMEM((_NEG_ROWS // 128, 128, 128), jnp.float32),
            pltpu.SemaphoreType.DMA,
        ],
    )


@functools.cache
def _make_sc_pos():
    return pl.kernel(
        _sc_pos_body,
        mesh=plsc.VectorSubcoreMesh(core_axis_name="c", subcore_axis_name="s"),
        out_type=[
            jax.ShapeDtypeStruct((_NTILES, _POS_CHUNKS, 128, 16),
                                 jnp.float32),
        ],
        scratch_types=[
            pltpu.VMEM((_PPT,), jnp.int32),
            pltpu.VMEM((_POS_CHUNKS, 128), jnp.int32),
            pltpu.VMEM((_POS_CHUNKS, 128, 16), jnp.float32),
            pltpu.SemaphoreType.DMA,
        ],
        compiler_params=pltpu.CompilerParams(use_tc_tiling_on_sc=False),
    )


_GRID = 8
_NB = B * B // 128 // _GRID
_PB = B * L_POS * 16 // 128 // _GRID


def _loss_body(gn_ref, gp_ref, out_ref, acc_ref):
    step = pl.program_id(0)

    xn = gn_ref[...]
    sn = jax.nn.sigmoid(xn)
    neg_sum = jnp.sum(-jnp.log(1.0 - sn + 1e-10))

    praw = gp_ref[...]
    rr = lax.broadcasted_iota(jnp.int32, praw.shape, 0) + step * _PB
    ll = lax.broadcasted_iota(jnp.int32, praw.shape, 1)
    e = rr * 8 + ll // 16
    tgt = (e // L_POS) & 15
    mask = (ll & 15) == tgt
    gp = -jnp.log(jax.nn.sigmoid(praw) + 1e-10)
    pos_sum = jnp.sum(jnp.where(mask, gp, 0.0))

    part = neg_sum / B + pos_sum / (L_POS * B)

    @pl.when(step == 0)
    def _():
        acc_ref[0] = 0.0

    acc_ref[0] += part

    @pl.when(step == _GRID - 1)
    def _():
        out_ref[...] = jnp.reshape(acc_ref[0], (1, 1))


def _loss_reduce(gneg, gpos):
    return pl.pallas_call(
        _loss_body,
        grid=(_GRID,),
        in_specs=[
            pl.BlockSpec((_NB, 128), lambda i: (i, 0)),
            pl.BlockSpec((_PB, 128), lambda i: (i, 0)),
        ],
        out_specs=pl.BlockSpec((1, 1), lambda i: (0, 0)),
        out_shape=jax.ShapeDtypeStruct((1, 1), jnp.float32),
        scratch_shapes=[pltpu.SMEM((1,), jnp.float32)],
    )(gneg, gpos)


def kernel(outputs, labels, sessions):
    cand = _cand_pool()
    neg = _neg_select(labels, sessions, cand).reshape(B)
    view = (jnp.transpose(outputs)
            .reshape(NUM_CLASSES // 8, 8, 8, 128)
            .transpose(0, 2, 1, 3)
            .reshape(NUM_CLASSES * 8, 128))
    view16 = view.reshape(NUM_CLASSES * 64, 16)
    (gneg,) = _make_sc_neg()(view, neg)
    (gpos,) = _make_sc_pos()(view16, labels.reshape(-1))
    loss = _loss_reduce(gneg.reshape(B * B // 128, 128),
                        gpos.reshape(B * L_POS * 16 // 128, 128))
    return loss[0, 0]


# speedup vs baseline: 16.0964x; 1.0501x over previous
import functools

import jax
import jax.numpy as jnp
from jax import lax
from jax.experimental import pallas as pl
from jax.experimental.pallas import tpu as pltpu
from jax.experimental.pallas import tpu_sc as plsc

B = 1024
L_POS = 20
L_SESS = 50
NUM_CLASSES = 100000
NEG_CAND = 256

_NTILES = 32
_COLS_PER_TILE = B // _NTILES
_ROWS_PER_TILE = B // _NTILES
_PPT = _ROWS_PER_TILE * L_POS
_NEG_ROWS = _COLS_PER_TILE * 8
_POS_CHUNKS = _PPT // 128


def _cand_pool():
    return jax.vmap(lambda k: jax.random.randint(k, (NEG_CAND,), 0, NUM_CLASSES))(
        jax.random.split(jax.random.key(42), B)
    )


try:
    import numpy as _np
    _CAND_CONST = _np.asarray(_cand_pool())
except Exception:
    _CAND_CONST = None


def _neg_select_body(lab_ref, ses_ref, cand_ref, neg_ref):
    lab = lab_ref[...]
    ses = ses_ref[...]
    cnd = cand_ref[...]
    bad = jnp.zeros(cnd.shape, jnp.bool_)
    for t in range(L_POS):
        bad = bad | (cnd == lab[:, t][:, None])
    for t in range(L_SESS):
        bad = bad | (cnd == ses[:, t][:, None])
    ii = lax.broadcasted_iota(jnp.int32, cnd.shape, 1)
    score = jnp.where(bad, jnp.int32(1 << 20), ii)
    first = jnp.min(score, axis=1, keepdims=True)
    sel = jnp.where(first >= NEG_CAND, 0, first)
    neg_ref[...] = jnp.sum(jnp.where(ii == sel, cnd, 0), axis=1, keepdims=True)


def _neg_select(labels, sessions, cand):
    blk = 128
    return pl.pallas_call(
        _neg_select_body,
        grid=(B // blk,),
        in_specs=[
            pl.BlockSpec((blk, L_POS), lambda i: (i, 0)),
            pl.BlockSpec((blk, L_SESS), lambda i: (i, 0)),
            pl.BlockSpec((blk, NEG_CAND), lambda i: (i, 0)),
        ],
        out_specs=pl.BlockSpec((blk, 1), lambda i: (i, 0)),
        out_shape=jax.ShapeDtypeStruct((B, 1), jnp.int32),
    )(labels, sessions, cand)


def _sc_gather_body(view16, neg_hbm, labflat, gneg_hbm, gpos_hbm,
                    negw_v, lab_v, idx_v, dat_v, pidx_v, pdat_v, sem):
    wid = lax.axis_index("s") * 2 + lax.axis_index("c")

    pltpu.sync_copy(neg_hbm.at[pl.ds(wid * _COLS_PER_TILE, _COLS_PER_TILE)],
                    negw_v.at[pl.ds(0, _COLS_PER_TILE)])
    pltpu.sync_copy(labflat.at[pl.ds(wid * _PPT, _PPT)], lab_v)

    three = jnp.full((16,), 3, jnp.int32)
    four = jnp.full((16,), 4, jnp.int32)
    six = jnp.full((16,), 6, jnp.int32)
    seven = jnp.full((16,), 7, jnp.int32)
    nine = jnp.full((16,), 9, jnp.int32)
    m127 = jnp.full((16,), 127, jnp.int32)
    lane = lax.iota(jnp.int32, 16)

    for jc in range(_COLS_PER_TILE // 16):
        c = negw_v[pl.ds(jc * 16, 16)]
        base8 = (lax.shift_left(lax.shift_right_logical(c, three), nine)
                 + lax.shift_left(c & seven, three))
        for it in range(8):
            for sub in range(8):
                p0 = (it * 8 + sub) * _COLS_PER_TILE + jc * 16
                idx_v[p0 // 128, pl.ds(p0 % 128, 16)] = (
                    base8 + jnp.full((16,), it * 64 + sub, jnp.int32))

    for c40 in range(_PPT // 16):
        k0 = c40 * 16
        r0 = k0 // L_POS
        bound = (r0 + 1) * L_POS
        inc = jnp.where(lane >= jnp.full((16,), bound - k0, jnp.int32),
                        jnp.full((16,), 1, jnp.int32),
                        jnp.full((16,), 0, jnp.int32))
        ivec = jnp.full((16,), wid * _ROWS_PER_TILE + r0, jnp.int32) + inc
        cvec = lab_v[pl.ds(k0, 16)]
        rvec = (lax.shift_left(lax.shift_right_logical(cvec, three), six)
                + lax.shift_left(lax.shift_right_logical(ivec, seven), three)
                + (cvec & seven))
        gvec = (lax.shift_left(rvec, three)
                + lax.shift_right_logical(ivec & m127, four))
        pidx_v[c40 // 8, pl.ds((c40 % 8) * 16, 16)] = gvec

    cps = [pltpu.async_copy(view16.at[idx_v.at[q]], dat_v.at[q], sem)
           for q in range(_NEG_ROWS * 8 // 128)]
    cps += [pltpu.async_copy(view16.at[pidx_v.at[q]], pdat_v.at[q], sem)
            for q in range(_POS_CHUNKS)]
    for cp in cps:
        cp.wait()
    pltpu.sync_copy(dat_v, gneg_hbm.at[wid])
    pltpu.sync_copy(pdat_v, gpos_hbm.at[wid])


@functools.cache
def _make_sc_gather():
    return pl.kernel(
        _sc_gather_body,
        mesh=plsc.VectorSubcoreMesh(core_axis_name="c", subcore_axis_name="s"),
        out_type=[
            jax.ShapeDtypeStruct((_NTILES, _NEG_ROWS * 8 // 128, 128, 16),
                                 jnp.float32),
            jax.ShapeDtypeStruct((_NTILES, _POS_CHUNKS, 128, 16),
                                 jnp.float32),
        ],
        scratch_types=[
            pltpu.VMEM((128,), jnp.int32),
            pltpu.VMEM((_PPT,), jnp.int32),
            pltpu.VMEM((_NEG_ROWS * 8 // 128, 128), jnp.int32),
            pltpu.VMEM((_NEG_ROWS * 8 // 128, 128, 16), jnp.float32),
            pltpu.VMEM((_POS_CHUNKS, 128), jnp.int32),
            pltpu.VMEM((_POS_CHUNKS, 128, 16), jnp.float32),
            pltpu.SemaphoreType.DMA,
        ],
        compiler_params=pltpu.CompilerParams(use_tc_tiling_on_sc=False),
    )


_GRID = 8
_NB = B * B // 128 // _GRID
_PB = B * L_POS * 16 // 128 // _GRID


def _loss_body(gn_ref, gp_ref, out_ref, acc_ref):
    step = pl.program_id(0)

    xn = gn_ref[...]
    sn = jax.nn.sigmoid(xn)
    neg_sum = jnp.sum(-jnp.log(1.0 - sn + 1e-10))

    praw = gp_ref[...]
    rr = lax.broadcasted_iota(jnp.int32, praw.shape, 0) + step * _PB
    ll = lax.broadcasted_iota(jnp.int32, praw.shape, 1)
    e = rr * 8 + ll // 16
    tgt = (e // L_POS) & 15
    mask = (ll & 15) == tgt
    gp = -jnp.log(jax.nn.sigmoid(praw) + 1e-10)
    pos_sum = jnp.sum(jnp.where(mask, gp, 0.0))

    part = neg_sum / B + pos_sum / (L_POS * B)

    @pl.when(step == 0)
    def _():
        acc_ref[0] = 0.0

    acc_ref[0] += part

    @pl.when(step == _GRID - 1)
    def _():
        out_ref[...] = jnp.reshape(acc_ref[0], (1, 1))


def _loss_reduce(gneg, gpos):
    return pl.pallas_call(
        _loss_body,
        grid=(_GRID,),
        in_specs=[
            pl.BlockSpec((_NB, 128), lambda i: (i, 0)),
            pl.BlockSpec((_PB, 128), lambda i: (i, 0)),
        ],
        out_specs=pl.BlockSpec((1, 1), lambda i: (0, 0)),
        out_shape=jax.ShapeDtypeStruct((1, 1), jnp.float32),
        scratch_shapes=[pltpu.SMEM((1,), jnp.float32)],
    )(gneg, gpos)


def kernel(outputs, labels, sessions):
    cand = (jnp.asarray(_CAND_CONST) if _CAND_CONST is not None
            else _cand_pool())
    neg = _neg_select(labels, sessions, cand).reshape(B)
    view16 = (jnp.transpose(outputs)
              .reshape(NUM_CLASSES // 8, 8, 8, 128)
              .transpose(0, 2, 1, 3)
              .reshape(NUM_CLASSES * 64, 16))
    gneg, gpos = _make_sc_gather()(view16, neg, labels.reshape(-1))
    loss = _loss_reduce(gneg.reshape(B * B // 128, 128),
                        gpos.reshape(B * L_POS * 16 // 128, 128))
    return loss[0, 0]


# speedup vs baseline: 17.3519x; 1.0780x over previous
import functools

import jax
import jax.numpy as jnp
from jax import lax
from jax.experimental import pallas as pl
from jax.experimental.pallas import tpu as pltpu
from jax.experimental.pallas import tpu_sc as plsc

B = 1024
L_POS = 20
L_SESS = 50
NUM_CLASSES = 100000
NEG_CAND = 256

_NTILES = 32
_COLS_PER_TILE = B // _NTILES
_ROWS_PER_TILE = B // _NTILES
_PPT = _ROWS_PER_TILE * L_POS
_NEG_ROWS = _COLS_PER_TILE * 8
_POS_CHUNKS = _PPT // 128


def _cand_pool():
    return jax.vmap(lambda k: jax.random.randint(k, (NEG_CAND,), 0, NUM_CLASSES))(
        jax.random.split(jax.random.key(42), B)
    )


try:
    import numpy as _np
    _CAND_CONST = _np.asarray(_cand_pool())
except Exception:
    _CAND_CONST = None


def _neg_select_body(lab_ref, ses_ref, cand_ref, neg_ref):
    lab = lab_ref[...]
    ses = ses_ref[...]
    cnd = cand_ref[...]
    acc = cnd ^ lab[:, 0][:, None]
    for t in range(1, L_POS):
        acc = jnp.minimum(acc, cnd ^ lab[:, t][:, None])
    for t in range(L_SESS):
        acc = jnp.minimum(acc, cnd ^ ses[:, t][:, None])
    bad = acc == 0
    ii = lax.broadcasted_iota(jnp.int32, cnd.shape, 1)
    score = jnp.where(bad, jnp.int32(1 << 20), ii)
    first = jnp.min(score, axis=1, keepdims=True)
    sel = jnp.where(first >= NEG_CAND, 0, first)
    neg_ref[...] = jnp.sum(jnp.where(ii == sel, cnd, 0), axis=1, keepdims=True)


def _neg_select(labels, sessions, cand):
    blk = 128
    return pl.pallas_call(
        _neg_select_body,
        grid=(B // blk,),
        in_specs=[
            pl.BlockSpec((blk, L_POS), lambda i: (i, 0)),
            pl.BlockSpec((blk, L_SESS), lambda i: (i, 0)),
            pl.BlockSpec((blk, NEG_CAND), lambda i: (i, 0)),
        ],
        out_specs=pl.BlockSpec((blk, 1), lambda i: (i, 0)),
        out_shape=jax.ShapeDtypeStruct((B, 1), jnp.int32),
    )(labels, sessions, cand)


def _sc_gather_body(view16, neg_hbm, labflat, gneg_hbm, gpos_hbm,
                    negw_v, lab_v, idx_v, dat_v, pidx_v, pdat_v, sem):
    wid = lax.axis_index("s") * 2 + lax.axis_index("c")

    pltpu.sync_copy(neg_hbm.at[pl.ds(wid * _COLS_PER_TILE, _COLS_PER_TILE)],
                    negw_v.at[pl.ds(0, _COLS_PER_TILE)])
    pltpu.sync_copy(labflat.at[pl.ds(wid * _PPT, _PPT)], lab_v)

    three = jnp.full((16,), 3, jnp.int32)
    four = jnp.full((16,), 4, jnp.int32)
    six = jnp.full((16,), 6, jnp.int32)
    seven = jnp.full((16,), 7, jnp.int32)
    nine = jnp.full((16,), 9, jnp.int32)
    m127 = jnp.full((16,), 127, jnp.int32)
    lane = lax.iota(jnp.int32, 16)

    for jc in range(_COLS_PER_TILE // 16):
        c = negw_v[pl.ds(jc * 16, 16)]
        base8 = (lax.shift_left(lax.shift_right_logical(c, three), nine)
                 + lax.shift_left(c & seven, three))
        for it in range(8):
            for sub in range(8):
                p0 = (it * 8 + sub) * _COLS_PER_TILE + jc * 16
                idx_v[p0 // 128, pl.ds(p0 % 128, 16)] = (
                    base8 + jnp.full((16,), it * 64 + sub, jnp.int32))

    for c40 in range(_PPT // 16):
        k0 = c40 * 16
        r0 = k0 // L_POS
        bound = (r0 + 1) * L_POS
        inc = jnp.where(lane >= jnp.full((16,), bound - k0, jnp.int32),
                        jnp.full((16,), 1, jnp.int32),
                        jnp.full((16,), 0, jnp.int32))
        ivec = jnp.full((16,), wid * _ROWS_PER_TILE + r0, jnp.int32) + inc
        cvec = lab_v[pl.ds(k0, 16)]
        rvec = (lax.shift_left(lax.shift_right_logical(cvec, three), six)
                + lax.shift_left(lax.shift_right_logical(ivec, seven), three)
                + (cvec & seven))
        gvec = (lax.shift_left(rvec, three)
                + lax.shift_right_logical(ivec & m127, four))
        pidx_v[c40 // 8, pl.ds((c40 % 8) * 16, 16)] = gvec

    cps = [pltpu.async_copy(view16.at[idx_v.at[q]], dat_v.at[q], sem)
           for q in range(_NEG_ROWS * 8 // 128)]
    cps += [pltpu.async_copy(view16.at[pidx_v.at[q]], pdat_v.at[q], sem)
            for q in range(_POS_CHUNKS)]
    for cp in cps:
        cp.wait()
    pltpu.sync_copy(dat_v, gneg_hbm.at[wid])
    pltpu.sync_copy(pdat_v, gpos_hbm.at[wid])


@functools.cache
def _make_sc_gather():
    return pl.kernel(
        _sc_gather_body,
        mesh=plsc.VectorSubcoreMesh(core_axis_name="c", subcore_axis_name="s"),
        out_type=[
            jax.ShapeDtypeStruct((_NTILES, _NEG_ROWS * 8 // 128, 128, 16),
                                 jnp.float32),
            jax.ShapeDtypeStruct((_NTILES, _POS_CHUNKS, 128, 16),
                                 jnp.float32),
        ],
        scratch_types=[
            pltpu.VMEM((128,), jnp.int32),
            pltpu.VMEM((_PPT,), jnp.int32),
            pltpu.VMEM((_NEG_ROWS * 8 // 128, 128), jnp.int32),
            pltpu.VMEM((_NEG_ROWS * 8 // 128, 128, 16), jnp.float32),
            pltpu.VMEM((_POS_CHUNKS, 128), jnp.int32),
            pltpu.VMEM((_POS_CHUNKS, 128, 16), jnp.float32),
            pltpu.SemaphoreType.DMA,
        ],
        compiler_params=pltpu.CompilerParams(use_tc_tiling_on_sc=False),
    )


_GRID = 8
_NB = B * B // 128 // _GRID
_PB = B * L_POS * 16 // 128 // _GRID


def _loss_body(gn_ref, gp_ref, out_ref, acc_ref):
    step = pl.program_id(0)

    xn = gn_ref[...]
    sn = jax.nn.sigmoid(xn)
    neg_sum = jnp.sum(-jnp.log(1.0 - sn + 1e-10))

    praw = gp_ref[...]
    rr = lax.broadcasted_iota(jnp.int32, praw.shape, 0) + step * _PB
    ll = lax.broadcasted_iota(jnp.int32, praw.shape, 1)
    e = rr * 8 + ll // 16
    tgt = (e // L_POS) & 15
    mask = (ll & 15) == tgt
    gp = -jnp.log(jax.nn.sigmoid(praw) + 1e-10)
    pos_sum = jnp.sum(jnp.where(mask, gp, 0.0))

    part = neg_sum / B + pos_sum / (L_POS * B)

    @pl.when(step == 0)
    def _():
        acc_ref[0] = 0.0

    acc_ref[0] += part

    @pl.when(step == _GRID - 1)
    def _():
        out_ref[...] = jnp.reshape(acc_ref[0], (1, 1))


def _loss_reduce(gneg, gpos):
    return pl.pallas_call(
        _loss_body,
        grid=(_GRID,),
        in_specs=[
            pl.BlockSpec((_NB, 128), lambda i: (i, 0)),
            pl.BlockSpec((_PB, 128), lambda i: (i, 0)),
        ],
        out_specs=pl.BlockSpec((1, 1), lambda i: (0, 0)),
        out_shape=jax.ShapeDtypeStruct((1, 1), jnp.float32),
        scratch_shapes=[pltpu.SMEM((1,), jnp.float32)],
    )(gneg, gpos)


def kernel(outputs, labels, sessions):
    cand = (jnp.asarray(_CAND_CONST) if _CAND_CONST is not None
            else _cand_pool())
    neg = _neg_select(labels, sessions, cand).reshape(B)
    view16 = (jnp.transpose(outputs)
              .reshape(NUM_CLASSES // 8, 8, 8, 128)
              .transpose(0, 2, 1, 3)
              .reshape(NUM_CLASSES * 64, 16))
    gneg, gpos = _make_sc_gather()(view16, neg, labels.reshape(-1))
    loss = _loss_reduce(gneg.reshape(B * B // 128, 128),
                        gpos.reshape(B * L_POS * 16 // 128, 128))
    return loss[0, 0]
